# baseline (speedup 1.0000x reference)
import jax
import jax.numpy as jnp
from jax.experimental import pallas as pl
from jax.experimental.pallas import tpu as pltpu

_BS = 512


def _dma_bcast_kernel(w_ref, o_ref, sems):
    s = pl.program_id(0)
    base = s * _BS
    batch = o_ref.shape[0]
    copies = [
        pltpu.make_async_copy(w_ref, o_ref.at[b, pl.ds(base, _BS), :], sems.at[b])
        for b in range(batch)
    ]
    for c in copies:
        c.start()
    for c in copies:
        c.wait()


def kernel(tokens, W_pos):
    batch, seq_len = tokens.shape
    d_model = W_pos.shape[1]
    grid = seq_len // _BS
    return pl.pallas_call(
        _dma_bcast_kernel,
        grid=(grid,),
        in_specs=[pl.BlockSpec((_BS, d_model), lambda s: (s, 0))],
        out_specs=pl.BlockSpec(memory_space=pltpu.MemorySpace.HBM),
        out_shape=jax.ShapeDtypeStruct((batch, seq_len, d_model), W_pos.dtype),
        scratch_shapes=[pltpu.SemaphoreType.DMA((4,))],
    )(W_pos)


# speedup vs baseline: 1.0046x; 1.0046x over previous
import jax
import jax.numpy as jnp
from jax.experimental import pallas as pl
from jax.experimental.pallas import tpu as pltpu

_BS = 512


def _dma_bcast_kernel(w_ref, o_ref, sems):
    s = pl.program_id(0)
    base = s * _BS
    batch = o_ref.shape[0]
    half = _BS // 2
    copies = [
        pltpu.make_async_copy(
            w_ref.at[pl.ds(h * half, half), :],
            o_ref.at[b, pl.ds(base + h * half, half), :],
            sems.at[b * 2 + h],
        )
        for b in range(batch)
        for h in range(2)
    ]
    for c in copies:
        c.start()
    for c in copies:
        c.wait()


def kernel(tokens, W_pos):
    batch, seq_len = tokens.shape
    d_model = W_pos.shape[1]
    grid = seq_len // _BS
    return pl.pallas_call(
        _dma_bcast_kernel,
        grid=(grid,),
        in_specs=[pl.BlockSpec((_BS, d_model), lambda s: (s, 0))],
        out_specs=pl.BlockSpec(memory_space=pltpu.MemorySpace.HBM),
        out_shape=jax.ShapeDtypeStruct((batch, seq_len, d_model), W_pos.dtype),
        scratch_shapes=[pltpu.SemaphoreType.DMA((8,))],
    )(W_pos)


# speedup vs baseline: 1.0089x; 1.0042x over previous
import jax
import jax.numpy as jnp
from jax.experimental import pallas as pl
from jax.experimental.pallas import tpu as pltpu

_BS = 512


def _make_body(batch, seq_len, d_model):
    nchunks = seq_len // _BS

    def body(w_hbm, o_hbm, buf, insem, outsem):
        def in_copy(i, slot):
            return pltpu.make_async_copy(
                w_hbm.at[pl.ds(i * _BS, _BS), :], buf.at[slot], insem.at[slot]
            )

        def out_copy(i, slot, b):
            return pltpu.make_async_copy(
                buf.at[slot],
                o_hbm.at[b, pl.ds(i * _BS, _BS), :],
                outsem.at[slot, b],
            )

        in_copy(0, 0).start()
        for i in range(nchunks):
            slot = i % 2
            if i + 1 < nchunks:
                if i >= 1:
                    for b in range(batch):
                        out_copy(i - 1, (i - 1) % 2, b).wait()
                in_copy(i + 1, (i + 1) % 2).start()
            in_copy(i, slot).wait()
            for b in range(batch):
                out_copy(i, slot, b).start()
        for i in (nchunks - 2, nchunks - 1):
            for b in range(batch):
                out_copy(i, i % 2, b).wait()

    return body


def kernel(tokens, W_pos):
    batch, seq_len = tokens.shape
    d_model = W_pos.shape[1]
    return pl.pallas_call(
        _make_body(batch, seq_len, d_model),
        in_specs=[pl.BlockSpec(memory_space=pltpu.MemorySpace.HBM)],
        out_specs=pl.BlockSpec(memory_space=pltpu.MemorySpace.HBM),
        out_shape=jax.ShapeDtypeStruct((batch, seq_len, d_model), W_pos.dtype),
        scratch_shapes=[
            pltpu.VMEM((2, _BS, d_model), W_pos.dtype),
            pltpu.SemaphoreType.DMA((2,)),
            pltpu.SemaphoreType.DMA((2, 4)),
        ],
    )(W_pos)


# speedup vs baseline: 1.0245x; 1.0155x over previous
import jax
import jax.numpy as jnp
from jax.experimental import pallas as pl
from jax.experimental.pallas import tpu as pltpu

_BS = 256


def _bcast_kernel(w_ref, o_ref):
    o_ref[...] = jnp.broadcast_to(w_ref[...][None], o_ref.shape)


def kernel(tokens, W_pos):
    batch, seq_len = tokens.shape
    d_model = W_pos.shape[1]
    grid = seq_len // _BS
    return pl.pallas_call(
        _bcast_kernel,
        grid=(grid,),
        in_specs=[pl.BlockSpec((_BS, d_model), lambda s: (s, 0))],
        out_specs=pl.BlockSpec((batch, _BS, d_model), lambda s: (0, s, 0)),
        out_shape=jax.ShapeDtypeStruct((batch, seq_len, d_model), W_pos.dtype),
        compiler_params=pltpu.CompilerParams(
            dimension_semantics=("parallel",),
        ),
    )(W_pos)
